# baseline (speedup 1.0000x reference)
import jax
import jax.numpy as jnp
from jax import lax
from jax.experimental import pallas as pl
from jax.experimental.pallas import tpu as pltpu
from jax.experimental.pallas import tpu_sc as plsc

XS = 256
NPIX = XS * XS
B = 8
N = 12000
F = 64
FC = 16
NQ = F // FC
NC = 2
NS = 16
L = 16
PW = 768
NCHUNK = PW // L
NJ = PW // 128
ACC_PER_TEC = 4104
ACC_ROWS = ACC_PER_TEC * NS
TRASH = NPIX
BPC = B // NC
ZROWS = 1026
COPY_PER_TEC = NPIX // NS


def _sc_body(pil_hbm, cf_hbm, ct_hbm, out_hbm,
             acc, vals_v, idx_v, cf_v, ct_v, zbuf, sem):
    cid = lax.axis_index("c")
    wid = lax.axis_index("s")

    zero16 = jnp.zeros((L,), jnp.float32)

    def _zb(i, carry):
        zbuf[i, :] = zero16
        return carry

    lax.fori_loop(0, ZROWS, _zb, 0)

    p0 = jnp.minimum(PW * wid, N - PW)
    p0 = pl.multiple_of(p0, 8)
    wbase = PW * wid
    lanes = lax.iota(jnp.int32, L)

    for bi in range(BPC):
        bg = cid * BPC + bi
        pltpu.sync_copy(cf_hbm.at[bg, pl.ds(pl.multiple_of(p0 * 3, 8), 3 * PW)],
                        cf_v)
        pltpu.sync_copy(ct_hbm.at[bg, pl.ds(p0, PW)], ct_v)

        for t in range(NCHUNK):
            gi = lanes + (t * L)
            c1 = plsc.load_gather(cf_v, [gi * 3 + 1])
            c2 = plsc.load_gather(cf_v, [gi * 3 + 2])
            ct = ct_v[pl.ds(t * L, L)]
            flat = c1 * XS + c2
            keep = jnp.logical_and(ct == 1, (p0 + gi) >= wbase)
            idx_v[t // 8, pl.ds((t % 8) * L, L)] = jnp.where(keep, flat, TRASH)

        for q in range(NQ):
            for z in range(4):
                pltpu.sync_copy(
                    zbuf,
                    acc.at[pl.ds(wid * ACC_PER_TEC + z * ZROWS, ZROWS)])
            plsc.subcore_barrier()

            pltpu.sync_copy(pil_hbm.at[bg, pl.ds(p0, PW), pl.ds(q * FC, FC)],
                            vals_v)

            def _ser(k, carry):
                @pl.when(wid == k)
                def _scatter():
                    cps = [
                        pltpu.async_copy(vals_v.at[pl.ds(j * 128, 128)],
                                         acc.at[idx_v.at[j]], sem, add=True)
                        for j in range(NJ)
                    ]
                    for cp in cps:
                        cp.wait()
                plsc.subcore_barrier()
                return carry

            lax.fori_loop(0, NS, _ser, 0)

            pltpu.sync_copy(
                acc.at[pl.ds(wid * COPY_PER_TEC, COPY_PER_TEC)],
                out_hbm.at[bg, q, pl.ds(wid * COPY_PER_TEC, COPY_PER_TEC)])
            plsc.subcore_barrier()


def _sc_scatter(pillars, coord_flat, contains):
    mesh = plsc.VectorSubcoreMesh(core_axis_name="c", subcore_axis_name="s",
                                  num_cores=NC, num_subcores=NS)
    return pl.kernel(
        _sc_body,
        out_type=jax.ShapeDtypeStruct((B, NQ, NPIX, FC), jnp.float32),
        mesh=mesh,
        compiler_params=pltpu.CompilerParams(use_tc_tiling_on_sc=False,
                                             needs_layout_passes=False),
        scratch_types=[
            pltpu.VMEM_SHARED((ACC_ROWS, FC), jnp.float32),
            pltpu.VMEM((PW, FC), jnp.float32),
            pltpu.VMEM((NJ, 128), jnp.int32),
            pltpu.VMEM((3 * PW,), jnp.int32),
            pltpu.VMEM((PW,), jnp.int32),
            pltpu.VMEM((ZROWS, FC), jnp.float32),
            pltpu.SemaphoreType.DMA,
        ],
    )(pillars, coord_flat, contains)


_TPIX = 8192


def _tr_body(x_ref, o_ref):
    o_ref[0] = jnp.transpose(x_ref[0, 0], (1, 0))


def _transpose_tc(inter):
    return pl.pallas_call(
        _tr_body,
        grid=(B, NQ, NPIX // _TPIX),
        in_specs=[pl.BlockSpec((1, 1, _TPIX, FC), lambda b, q, s: (b, q, s, 0))],
        out_specs=pl.BlockSpec((1, FC, _TPIX), lambda b, q, s: (b, q, s)),
        out_shape=jax.ShapeDtypeStruct((B, F, NPIX), jnp.float32),
    )(inter)


def kernel(pillars, coord, contains_pillars):
    coord_flat = coord.reshape(B, N * 3)
    inter = _sc_scatter(pillars, coord_flat, contains_pillars)
    out3 = _transpose_tc(inter)
    return out3.reshape(B, F, XS, XS)


# speedup vs baseline: 2.9062x; 2.9062x over previous
import jax
import jax.numpy as jnp
from jax import lax
from jax.experimental import pallas as pl
from jax.experimental.pallas import tpu as pltpu
from jax.experimental.pallas import tpu_sc as plsc

XS = 256
NPIX = XS * XS
B = 8
N = 12000
F = 64
FC = 16
NQ = F // FC
NC = 2
NS = 16
L = 16
BPC = B // NC
OWN = NPIX // NS
OWNX = OWN // XS
TRASH = OWN
FCH = 1536
NFC = 8
CAP = 12288


def _sc_body(pil16, cf_hbm, ct_hbm, out_hbm,
             acc, bounce, idxbuf, cfc_v, ctc_v, sidx, gbl, sem):
    cid = lax.axis_index("c")
    wid = lax.axis_index("s")
    lane = lax.iota(jnp.int32, L)
    zero16 = jnp.zeros((L,), jnp.float32)
    zero16i = jnp.zeros((L,), jnp.int32)
    trash16 = jnp.full((L,), TRASH, jnp.int32)
    base_lo = OWN * wid

    def _batch(bi, carry):
        bg = cid * BPC + bi

        off = jnp.int32(0)
        for fc in range(NFC):
            p0c = min(FCH * fc, N - FCH)
            pltpu.sync_copy(cf_hbm.at[bg, pl.ds(3 * p0c, 3 * FCH)], cfc_v)
            pltpu.sync_copy(ct_hbm.at[bg, pl.ds(p0c, FCH)], ctc_v)

            def _chunk16(t, off, p0c=p0c, fc=fc):
                gi = lane + t * L
                c1 = plsc.load_gather(cfc_v, [gi * 3 + 1])
                c2 = plsc.load_gather(cfc_v, [gi * 3 + 2])
                ct = ctc_v[pl.ds(t * L, L)]
                local = c1 * XS + c2 - base_lo
                gp = p0c + gi
                keep = ((ct == 1) & (local >= 0) & (local < OWN)
                        & (gp >= FCH * fc))
                plsc.store_compressed(sidx.at[pl.ds(off, L)], local,
                                      mask=keep)
                plsc.store_compressed(gbl.at[pl.ds(off, L)],
                                      (gp + bg * N) * NQ, mask=keep)
                cnt = plsc.all_reduce_population_count(keep)
                return off + jnp.max(cnt)

            off = lax.fori_loop(0, FCH // L, _chunk16, off)
        n_w = off

        w0 = (n_w // L) * L
        live = lane < n_w - w0
        gbl[pl.ds(w0, L)] = jnp.where(live, gbl[pl.ds(w0, L)], 0)
        sidx[pl.ds(w0, L)] = jnp.where(live, sidx[pl.ds(w0, L)], trash16)
        for k in range(1, 9):
            gbl[pl.ds(w0 + k * L, L)] = zero16i
            sidx[pl.ds(w0 + k * L, L)] = trash16

        nch = (n_w + 127) >> 7

        def _fchunk(q, carry):
            def _zero(i, c):
                for f in range(FC):
                    acc[f, i >> 4, pl.ds((i & 15) * L, L)] = zero16
                return c
            lax.fori_loop(0, 256, _zero, 0)

            def _build_start(k, par):
                for s in range(8):
                    idxbuf[par, pl.ds(s * L, L)] = (
                        gbl[pl.ds(k * 128 + s * L, L)] + q)
                pltpu.async_copy(pil16.at[idxbuf.at[par]],
                                 bounce.at[pl.ds(par * 128, 128)], sem)

            @pl.when(nch > 0)
            def _prologue():
                _build_start(jnp.int32(0), jnp.int32(0))

            def _qloop(k, carry):
                par = k & 1
                pltpu.make_async_copy(
                    pil16.at[idxbuf.at[par]],
                    bounce.at[pl.ds(par * 128, 128)], sem).wait()

                @pl.when(k + 1 < nch)
                def _next():
                    _build_start(k + 1, 1 - par)

                def _group(g, c):
                    sv = sidx[pl.ds(k * 128 + g * L, L)]
                    xlv = sv >> 8
                    yv = sv & 255
                    for rr in range(L):
                        vals = bounce[par * 128 + g * L + rr, :]
                        xl = jnp.full((L,), xlv[rr], jnp.int32)
                        y = jnp.full((L,), yv[rr], jnp.int32)
                        plsc.addupdate_scatter(acc, [lane, xl, y], vals)
                    return c
                lax.fori_loop(0, 8, _group, 0)
                return carry
            lax.fori_loop(0, nch, _qloop, 0)

            pltpu.sync_copy(
                acc.at[:, pl.ds(0, OWNX), :],
                out_hbm.at[bg, pl.ds(q * FC, FC), pl.ds(wid * OWNX, OWNX), :])
            return carry
        lax.fori_loop(0, NQ, _fchunk, 0)
        return carry

    lax.fori_loop(0, BPC, _batch, 0)


def _sc_scatter(pil16, coord_flat, contains):
    mesh = plsc.VectorSubcoreMesh(core_axis_name="c", subcore_axis_name="s",
                                  num_cores=NC, num_subcores=NS)
    return pl.kernel(
        _sc_body,
        out_type=jax.ShapeDtypeStruct((B, F, XS, XS), jnp.float32),
        mesh=mesh,
        compiler_params=pltpu.CompilerParams(use_tc_tiling_on_sc=False,
                                             needs_layout_passes=False),
        scratch_types=[
            pltpu.VMEM((FC, OWNX + 1, XS), jnp.float32),
            pltpu.VMEM((256, FC), jnp.float32),
            pltpu.VMEM((2, 128), jnp.int32),
            pltpu.VMEM((3 * FCH,), jnp.int32),
            pltpu.VMEM((FCH,), jnp.int32),
            pltpu.VMEM((CAP,), jnp.int32),
            pltpu.VMEM((CAP,), jnp.int32),
            pltpu.SemaphoreType.DMA,
        ],
    )(pil16, coord_flat, contains)


def kernel(pillars, coord, contains_pillars):
    pil16 = pillars.reshape(B * N * NQ, FC)
    coord_flat = coord.reshape(B, N * 3)
    return _sc_scatter(pil16, coord_flat, contains_pillars)
